# baseline (speedup 1.0000x reference)
import functools
import jax
import jax.numpy as jnp
from jax import lax
from jax.experimental import pallas as pl
from jax.experimental.pallas import tpu as pltpu
from jax.experimental.pallas import tpu_sc as plsc

_W = 64
_H = 64
_B = 8
_J = 17
_BJ = _B * _J
_P = _W * _H
_NX = 8
_NSC = 4
_SCW = _P // _NSC

_NC = 2
_NS = 16
_NWORK = _NC * _NS
_UNITS = 2 * _W * _NSC
_UPW = _UNITS // _NWORK


def _sc_body(abc_hbm, c_hbm, abc_v, cloc_v, sem):
    wid = lax.axis_index("s") * _NC + lax.axis_index("c")
    ones = jnp.ones((16,), jnp.float32)
    zrow = jnp.zeros((16,), jnp.float32)
    lane = lax.broadcasted_iota(jnp.int32, (16,), 0)

    def unit(j, carry):
        u = wid * _UPW + j
        f = u // (_W * _NSC)
        rem = u % (_W * _NSC)
        xx = rem // _NSC
        sc = rem % _NSC
        xf = xx.astype(jnp.float32)

        pltpu.sync_copy(abc_hbm.at[f, sc], abc_v)

        def zrow_loop(r, carry2):
            for g in range(16):
                cloc_v[pl.ds(r * 256 + g * 16, 16)] = zrow
            return carry2
        lax.fori_loop(0, _H * _SCW // 256, zrow_loop, 0)

        def grp(g, carry2):
            col = g * 16 + lane
            for b in range(_B):
                a = abc_v[0, b, pl.ds(g * 16, 16)]
                bb = abc_v[1, b, pl.ds(g * 16, 16)]
                cc = abc_v[2, b, pl.ds(g * 16, 16)]
                yv = -(a * xf + cc) / bb
                valid = (yv >= 1.0) & (yv < 63.0)
                kc = jnp.where(valid, yv, 1.0).astype(jnp.int32)
                plsc.addupdate_scatter(cloc_v, [kc * _SCW + col], ones, mask=valid)
            return carry2
        lax.fori_loop(0, _SCW // 16, grp, 0)

        pltpu.sync_copy(cloc_v, c_hbm.at[f, xx, sc])
        return carry
    lax.fori_loop(0, _UPW, unit, 0)


def _build_counts(abc_sc):
    mesh = plsc.VectorSubcoreMesh(core_axis_name="c", subcore_axis_name="s")
    k = functools.partial(
        pl.kernel,
        out_type=jax.ShapeDtypeStruct((2, _W, _NSC, _H * _SCW), jnp.float32),
        mesh=mesh,
        scratch_types=[
            pltpu.VMEM((3, _B, _SCW), jnp.float32),
            pltpu.VMEM((_H * _SCW,), jnp.float32),
            pltpu.SemaphoreType.DMA,
        ],
        compiler_params=pltpu.CompilerParams(needs_layout_passes=False),
    )(_sc_body)
    return k(abc_sc)


def _tc_body(c_ref, xpad_ref, wt_ref, out_ref, hi_ref, lo_ref):
    xq = pl.program_id(1)

    @pl.when(xq == 0)
    def _():
        acc = jnp.zeros((_W, _H, _BJ), jnp.float32)
        for dy in range(3):
            for dx in range(3):
                w = wt_ref[0, dy, dx, :]
                acc += xpad_ref[dx:dx + _W, dy:dy + _H, :] * w[None, None, :]
        hi = acc.astype(jnp.bfloat16)
        hi_ref[...] = hi
        lo_ref[...] = (acc - hi.astype(jnp.float32)).astype(jnp.bfloat16)

    conv_hi = hi_ref[...].reshape(_P, _BJ)
    conv_lo = lo_ref[...].reshape(_P, _BJ)
    dn = (((1,), (0,)), ((), ()))
    zero_x = jnp.zeros((_NX * _H, _BJ), jnp.float32)
    for sc in range(_NSC):
        acc_sc = c_ref[0, :, sc, :].reshape(_NX * _H, _SCW).astype(jnp.bfloat16)
        chi = conv_hi[sc * _SCW:(sc + 1) * _SCW, :]
        clo = conv_lo[sc * _SCW:(sc + 1) * _SCW, :]
        zero_x = (zero_x
                  + lax.dot_general(acc_sc, chi, dn, preferred_element_type=jnp.float32)
                  + lax.dot_general(acc_sc, clo, dn, preferred_element_type=jnp.float32))
    m = 1.0 / (1.0 + jnp.exp(-zero_x))
    xrows = xpad_ref[pl.ds(xq * _NX + 1, _NX), 1:1 + _H, :]
    out_ref[0] = (xrows + m.reshape(_NX, _H, _BJ)) * 0.5


def kernel(x, y, F1, F2, W1, W2):
    del y
    x_t = jnp.transpose(x, (3, 2, 0, 1)).reshape(_W, _H, _BJ)
    x_pad = jnp.pad(x_t, ((1, 1), (1, 1), (0, 0)))

    def tile_w(Wf):
        return jnp.tile(jnp.transpose(Wf[:, 0], (1, 2, 0)), (1, 1, _B))
    wt = jnp.stack([tile_w(W2), tile_w(W1)])

    xs_p = jnp.repeat(jnp.arange(_W), _H)
    ys_p = jnp.tile(jnp.arange(_H), _W)
    px = jnp.stack([xs_p, ys_p, jnp.ones(_P, jnp.int32)], axis=1).astype(jnp.float32)
    abc = jnp.einsum('pk,bkm->bpm', px, jnp.stack([F1, F2]).reshape(2 * _B, 3, 3))
    abc = abc.reshape(2, _B, _P, 3)
    abc_sc = jnp.transpose(abc, (0, 3, 1, 2)).reshape(2, 3, _B, _NSC, _SCW)
    abc_sc = jnp.transpose(abc_sc, (0, 3, 1, 2, 4))

    counts = _build_counts(abc_sc)

    out = pl.pallas_call(
        _tc_body,
        grid=(2, _W // _NX),
        in_specs=[
            pl.BlockSpec((1, _NX, _NSC, _H * _SCW), lambda f, xq: (f, xq, 0, 0)),
            pl.BlockSpec((_W + 2, _H + 2, _BJ), lambda f, xq: (0, 0, 0)),
            pl.BlockSpec((1, 3, 3, _BJ), lambda f, xq: (f, 0, 0, 0)),
        ],
        out_specs=pl.BlockSpec((1, _NX, _H, _BJ), lambda f, xq: (f, xq, 0, 0)),
        out_shape=jax.ShapeDtypeStruct((2, _W, _H, _BJ), jnp.float32),
        scratch_shapes=[pltpu.VMEM((_W, _H, _BJ), jnp.bfloat16),
                        pltpu.VMEM((_W, _H, _BJ), jnp.bfloat16)],
    )(counts, x_pad, wt)

    def untile(o):
        return jnp.transpose(o.reshape(_W, _H, _B, _J), (2, 3, 1, 0))

    return (untile(out[0]), untile(out[1]), x, x)


# speedup vs baseline: 2.7015x; 2.7015x over previous
import jax
import jax.numpy as jnp
from jax import lax
from jax.experimental import pallas as pl
from jax.experimental.pallas import tpu as pltpu

_W = 64
_H = 64
_B = 8
_J = 17
_BJ = _B * _J
_P = _W * _H
_NX = 8


def _body(abc_ref, xpad_ref, wt_ref, out_ref, hi_ref, lo_ref):
    xq = pl.program_id(1)

    @pl.when(xq == 0)
    def _():
        acc = jnp.zeros((_W, _H, _BJ), jnp.float32)
        for dy in range(3):
            for dx in range(3):
                w = wt_ref[0, dy, dx, :]
                acc += xpad_ref[dx:dx + _W, dy:dy + _H, :] * w[None, None, :]
        hi = acc.astype(jnp.bfloat16)
        hi_ref[...] = hi
        lo_ref[...] = (acc - hi.astype(jnp.float32)).astype(jnp.bfloat16)

    yo = lax.broadcasted_iota(jnp.int32, (_H, _P), 0).astype(jnp.bfloat16)
    one = jnp.bfloat16(1.0)
    zero = jnp.bfloat16(0.0)

    accs = []
    for i in range(_NX):
        xf = (xq * _NX + i).astype(jnp.float32)
        acc = jnp.zeros((_H, _P), jnp.bfloat16)
        for b in range(_B):
            a = abc_ref[0, 0, b, :].reshape(1, _P)
            bb = abc_ref[0, 1, b, :].reshape(1, _P)
            cc = abc_ref[0, 2, b, :].reshape(1, _P)
            yv = -(a * xf + cc) / bb
            valid = (yv >= 1.0) & (yv < 63.0)
            kc = jnp.where(valid, jnp.floor(yv), 64.0).astype(jnp.bfloat16)
            acc += jnp.where(kc == yo, one, zero)
        accs.append(acc)
    acc_full = jnp.concatenate(accs, axis=0)

    conv_hi = hi_ref[...].reshape(_P, _BJ)
    conv_lo = lo_ref[...].reshape(_P, _BJ)
    dn = (((1,), (0,)), ((), ()))
    zero_x = (lax.dot_general(acc_full, conv_hi, dn, preferred_element_type=jnp.float32)
              + lax.dot_general(acc_full, conv_lo, dn, preferred_element_type=jnp.float32))
    m = 1.0 / (1.0 + jnp.exp(-zero_x))
    xrows = xpad_ref[pl.ds(xq * _NX + 1, _NX), 1:1 + _H, :]
    out_ref[0] = (xrows + m.reshape(_NX, _H, _BJ)) * 0.5


def kernel(x, y, F1, F2, W1, W2):
    del y
    x_t = jnp.transpose(x, (3, 2, 0, 1)).reshape(_W, _H, _BJ)
    x_pad = jnp.pad(x_t, ((1, 1), (1, 1), (0, 0)))
    def tile_w(Wf):
        return jnp.tile(jnp.transpose(Wf[:, 0], (1, 2, 0)), (1, 1, _B))
    wt = jnp.stack([tile_w(W2), tile_w(W1)])
    xs_p = jnp.repeat(jnp.arange(_W), _H)
    ys_p = jnp.tile(jnp.arange(_H), _W)
    px = jnp.stack([xs_p, ys_p, jnp.ones(_P, jnp.int32)], axis=1).astype(jnp.float32)
    abc = jnp.einsum('pk,bkm->bpm', px, jnp.stack([F1, F2]).reshape(2 * _B, 3, 3))
    abc = abc.reshape(2, _B, _P, 3)
    abc_t = jnp.transpose(abc, (0, 3, 1, 2))

    out = pl.pallas_call(
        _body,
        grid=(2, _W // _NX),
        in_specs=[
            pl.BlockSpec((1, 3, _B, _P), lambda f, xq: (f, 0, 0, 0)),
            pl.BlockSpec((_W + 2, _H + 2, _BJ), lambda f, xq: (0, 0, 0)),
            pl.BlockSpec((1, 3, 3, _BJ), lambda f, xq: (f, 0, 0, 0)),
        ],
        out_specs=pl.BlockSpec((1, _NX, _H, _BJ), lambda f, xq: (f, xq, 0, 0)),
        out_shape=jax.ShapeDtypeStruct((2, _W, _H, _BJ), jnp.float32),
        scratch_shapes=[pltpu.VMEM((_W, _H, _BJ), jnp.bfloat16),
                        pltpu.VMEM((_W, _H, _BJ), jnp.bfloat16)],
    )(abc_t, x_pad, wt)

    def untile(o):
        return jnp.transpose(o.reshape(_W, _H, _B, _J), (2, 3, 1, 0))

    x_out1 = untile(out[0])
    x_out2 = untile(out[1])
    return (x_out1, x_out2, x, x)
